# baseline (speedup 1.0000x reference)
import functools

import jax
import jax.numpy as jnp
from jax import lax
from jax.experimental import pallas as pl
from jax.experimental.pallas import tpu as pltpu
from jax.experimental.pallas import tpu_sc as plsc

_M = 32
_NCOLS = 33
_LANES = 16
_NUM_CORES = 2
_NUM_SUBCORES = 16
_NUM_WORKERS = _NUM_CORES * _NUM_SUBCORES


def _make_sc_call(batch):
    rows_per_worker = batch // _NUM_WORKERS
    groups = rows_per_worker // _LANES
    mesh = plsc.VectorSubcoreMesh(
        core_axis_name="c", subcore_axis_name="s",
        num_cores=_NUM_CORES, num_subcores=_NUM_SUBCORES)

    @functools.partial(
        pl.kernel,
        mesh=mesh,
        out_type=jax.ShapeDtypeStruct((batch,), jnp.int32),
        scratch_types=[
            pltpu.VMEM((_M, rows_per_worker), jnp.int32),
            pltpu.VMEM((_NCOLS, _NCOLS), jnp.int32),
            pltpu.VMEM((rows_per_worker,), jnp.int32),
        ],
        compiler_params=pltpu.CompilerParams(
            needs_layout_passes=False,
            disable_bounds_checks=True,
            disable_semaphore_checks=True,
        ),
    )
    def sc_rank(xt_hbm, comb_hbm, out_hbm, x_v, comb_v, out_v):
        wid = (lax.axis_index("s") * jnp.int32(_NUM_CORES)
               + lax.axis_index("c"))
        rbase = wid * jnp.int32(rows_per_worker)
        pltpu.sync_copy(comb_hbm, comb_v)
        pltpu.sync_copy(xt_hbm.at[:, pl.ds(rbase, rows_per_worker)], x_v)

        def group_body(g, carry):
            gbase = g * jnp.int32(_LANES)
            presum = jnp.zeros((_LANES,), jnp.int32)
            acc = jnp.zeros((_LANES,), jnp.int32)
            for i in range(_M):
                xi = x_v[i, pl.ds(gbase, _LANES)]
                row_i = jnp.full((_LANES,), _M - 1 - i, jnp.int32)
                cval = plsc.load_gather(comb_v, [row_i, presum])
                acc = acc + cval * xi
                presum = presum + xi
            out_v[pl.ds(gbase, _LANES)] = acc
            return carry

        lax.fori_loop(jnp.int32(0), jnp.int32(groups), group_body,
                      jnp.int32(0))
        pltpu.sync_copy(out_v, out_hbm.at[pl.ds(rbase, rows_per_worker)])

    return sc_rank


@jax.jit
def kernel(x, comb):
    batch = x.shape[0]
    xt32 = x.T.astype(jnp.int32)
    comb32r = comb.astype(jnp.int32)[:, ::-1]
    out32 = _make_sc_call(batch)(xt32, comb32r)
    return out32.astype(jnp.int64)
